# baseline (speedup 1.0000x reference)
import functools

import jax
import jax.numpy as jnp
from jax import lax
from jax.experimental import pallas as pl
from jax.experimental.pallas import tpu as pltpu
from jax.experimental.pallas import tpu_sc as plsc

_NC = 2
_NS = 16
_NW = _NC * _NS
_CH = 88
_NBUF = 4
_IR = _NBUF + 1

_SPLIT0 = 0.63

_B = 1000


def _sigmoid(v):
    return 1.0 / (1.0 + jnp.exp(-v))


def _pre_body(x_ref, wn_ref, ws_ref, wimp_ref, bimp_ref, msg_ref, xs_ref):
    x = x_ref[...]
    xn = jnp.dot(x, wn_ref[...], preferred_element_type=jnp.float32)
    logit = jnp.dot(xn, wimp_ref[...], preferred_element_type=jnp.float32) + bimp_ref[...]
    msg_ref[...] = _sigmoid(logit) * jnp.maximum(xn, 0.0)
    xs_ref[...] = jnp.dot(x, ws_ref[...], preferred_element_type=jnp.float32)


def _mid_body(parts_ref, xs_ref, b_ref, wn_ref, ws_ref, wimp_ref, bimp_ref,
              msg_ref, xs2_ref):
    p = parts_ref[...]
    h = jnp.maximum(p[0] + p[1] + xs_ref[...] + b_ref[...], 0.0)
    xn = jnp.dot(h, wn_ref[...], preferred_element_type=jnp.float32)
    logit = jnp.dot(xn, wimp_ref[...], preferred_element_type=jnp.float32) + bimp_ref[...]
    msg_ref[...] = _sigmoid(logit) * jnp.maximum(xn, 0.0)
    xs2_ref[...] = jnp.dot(h, ws_ref[...], preferred_element_type=jnp.float32)


def _post_body(parts_ref, xs_ref, b_ref, wm1_ref, bm1_ref, wm2_ref, bm2_ref,
               wm3_ref, bm3_ref, out_ref):
    p = parts_ref[...]
    h = jnp.maximum(p[0] + p[1] + xs_ref[...] + b_ref[...], 0.0)
    m = jnp.maximum(jnp.dot(h, wm1_ref[...], preferred_element_type=jnp.float32)
                    + bm1_ref[...], 0.0)
    m = jnp.maximum(jnp.dot(m, wm2_ref[...], preferred_element_type=jnp.float32)
                    + bm2_ref[...], 0.0)
    out_ref[...] = _sigmoid(
        jnp.dot(m, wm3_ref[...], preferred_element_type=jnp.float32)
        + bm3_ref[...])


def _full(shape):
    return pl.BlockSpec(shape, lambda i: (0,) * len(shape))


def _rows(shape):
    return pl.BlockSpec(shape, lambda i: (i,) + (0,) * (len(shape) - 1))


def _parts_spec(h):
    return pl.BlockSpec((_NC, _B, h), lambda i: (0, i, 0))


def _pre_call(x, Wn, Ws, Wimp, bimp):
    n, d = x.shape
    h = Wn.shape[1]
    grid = n // _B
    return pl.pallas_call(
        _pre_body,
        grid=(grid,),
        in_specs=[_rows((_B, d)), _full((d, h)), _full((d, h)),
                  _full((d, 1)), _full((1, 1))],
        out_specs=[_rows((_B, h)), _rows((_B, h))],
        out_shape=[jax.ShapeDtypeStruct((n, h), jnp.float32),
                   jax.ShapeDtypeStruct((n, h), jnp.float32)],
    )(x, Wn, Ws, Wimp.reshape(d, 1), bimp.reshape(1, 1))


def _mid_call(parts, xs, b, Wn, Ws, Wimp, bimp):
    n, h = xs.shape
    h2 = Wn.shape[1]
    grid = n // _B
    return pl.pallas_call(
        _mid_body,
        grid=(grid,),
        in_specs=[_parts_spec(h), _rows((_B, h)), _full((1, h)),
                  _full((h, h2)), _full((h, h2)), _full((h2, 1)), _full((1, 1))],
        out_specs=[_rows((_B, h2)), _rows((_B, h2))],
        out_shape=[jax.ShapeDtypeStruct((n, h2), jnp.float32),
                   jax.ShapeDtypeStruct((n, h2), jnp.float32)],
    )(parts, xs, b.reshape(1, h), Wn, Ws, Wimp.reshape(h, 1),
      bimp.reshape(1, 1))


def _post_call(parts, xs, b, Wm1, bm1, Wm2, bm2, Wm3, bm3):
    n, h = xs.shape
    d1, d2, d3 = Wm1.shape[1], Wm2.shape[1], Wm3.shape[1]
    grid = n // _B
    return pl.pallas_call(
        _post_body,
        grid=(grid,),
        in_specs=[_parts_spec(h), _rows((_B, h)), _full((1, h)),
                  _full((h, d1)), _full((1, d1)),
                  _full((d1, d2)), _full((1, d2)),
                  _full((d2, d3)), _full((1, d3))],
        out_specs=_rows((_B, d3)),
        out_shape=jax.ShapeDtypeStruct((n, d3), jnp.float32),
    )(parts, xs, b.reshape(1, h), Wm1, bm1.reshape(1, d1),
      Wm2, bm2.reshape(1, d2), Wm3, bm3.reshape(1, d3))


@functools.lru_cache(maxsize=None)
def _make_agg(q0, q1, acc_rows, h):
    rows_per_tile = acc_rows // _NS
    mesh = plsc.VectorSubcoreMesh(core_axis_name="c", subcore_axis_name="s")

    @functools.partial(
        pl.kernel,
        mesh=mesh,
        out_type=jax.ShapeDtypeStruct((_NC, acc_rows, h), jnp.float32),
        scratch_types=(
            [pltpu.VMEM((_CH, h), jnp.float32) for _ in range(_NBUF)]
            + [pltpu.VMEM((2, _CH), jnp.int32) for _ in range(_IR)]
            + [pltpu.VMEM_SHARED((acc_rows, h), jnp.float32)]
            + [pltpu.SemaphoreType.DMA for _ in range(2 * _NBUF + _IR)]
        ),
    )
    def agg(msg_hbm, srcb_hbm, dstb_hbm, zeros_hbm, out_hbm, *scr):
        rows = scr[:_NBUF]
        ibufs = scr[_NBUF:_NBUF + _IR]
        acc = scr[_NBUF + _IR]
        rsems = scr[_NBUF + _IR + 1:2 * _NBUF + _IR + 1]
        ssems = scr[2 * _NBUF + _IR + 1:3 * _NBUF + _IR + 1]
        isems = scr[3 * _NBUF + _IR + 1:]
        c = lax.axis_index("c")
        s = lax.axis_index("s")
        base = jnp.where(c == 0, s * q0, _NS * q0 + s * q1)
        my_n = jnp.where(c == 0, q0, q1)

        def wait_rows(b):
            pltpu.make_async_copy(msg_hbm.at[pl.ds(0, _CH)], rows[b],
                                  rsems[b]).wait()

        def wait_scatter(b):
            pltpu.make_async_copy(msg_hbm.at[pl.ds(0, _CH)], rows[b],
                                  ssems[b]).wait()

        def fetch_idx(j, ib):
            pltpu.async_copy(srcb_hbm.at[j], ibufs[ib].at[0], isems[ib])
            pltpu.async_copy(dstb_hbm.at[j], ibufs[ib].at[1], isems[ib])

        def gather(b, ib):
            pltpu.make_async_copy(srcb_hbm.at[0], ibufs[ib].at[0],
                                  isems[ib]).wait()
            pltpu.make_async_copy(srcb_hbm.at[0], ibufs[ib].at[1],
                                  isems[ib]).wait()
            pltpu.async_copy(msg_hbm.at[ibufs[ib].at[0]], rows[b], rsems[b])

        for k in range(_NBUF):
            @pl.when(k < my_n)
            def _(k=k):
                fetch_idx(base + k, k)
        pltpu.sync_copy(zeros_hbm, acc.at[pl.ds(s * rows_per_tile, rows_per_tile)])
        for k in range(_NBUF - 1):
            @pl.when(k < my_n)
            def _(k=k):
                gather(k, k)
        plsc.subcore_barrier()

        def slot(j, b, ib):
            pb = (b + _NBUF - 1) % _NBUF

            @pl.when((j >= 1) & (j - 1 < my_n))
            def _():
                wait_scatter(pb)

            @pl.when(j + _NBUF - 1 < my_n)
            def _():
                gather(pb, (ib + _NBUF - 1) % _IR)

            @pl.when(j < my_n)
            def _():
                wait_rows(b)
                pltpu.async_copy(rows[b], acc.at[ibufs[ib].at[1]], ssems[b],
                                 add=True)

            @pl.when(j + _NBUF < my_n)
            def _():
                fetch_idx(base + j + _NBUF, (ib + _NBUF) % _IR)

        unroll = _NBUF * _IR

        def body(g, carry):
            for u in range(unroll):
                j = unroll * g + u

                @pl.when(j < my_n + 1)
                def _():
                    slot(j, u % _NBUF, u % _IR)
            return carry

        lax.fori_loop(0, -(-(max(q0, q1) + 1) // unroll), body, 0)
        plsc.subcore_barrier()
        pltpu.sync_copy(acc.at[pl.ds(s * rows_per_tile, rows_per_tile)],
                        out_hbm.at[c, pl.ds(s * rows_per_tile, rows_per_tile)])

    return agg


def kernel(x, edge_index, Wn1, Ws1, Wimp1, bimp1, b1, Wn2, Ws2, Wimp2, bimp2,
           b2, Wm1, bm1, Wm2, bm2, Wm3, bm3):
    n, d = x.shape
    e = edge_index.shape[1]
    h = Wn1.shape[1]

    nchunks = -(-e // _CH)
    per_pair = -(-nchunks // _NS)
    q0 = max(1, min(per_pair - 1, round(per_pair * _SPLIT0)))
    q1 = per_pair - q0
    tot = per_pair * _NS
    ep = tot * _CH
    acc_rows = (_NS * 8) * (-(-(n + 1) // (_NS * 8)))

    src = edge_index[0].astype(jnp.int32)
    dst = edge_index[1].astype(jnp.int32)
    pad = ep - e
    srcb = jnp.concatenate([src, jnp.zeros((pad,), jnp.int32)]).reshape(
        tot, _CH)
    dstb = jnp.concatenate([dst, jnp.full((pad,), n, jnp.int32)]).reshape(
        tot, _CH)
    zeros = jnp.zeros((acc_rows // _NS, h), jnp.float32)

    agg_fn = _make_agg(q0, q1, acc_rows, h)

    msg1, xs1 = _pre_call(x, Wn1, Ws1, Wimp1, bimp1)
    parts1 = agg_fn(msg1, srcb, dstb, zeros)
    msg2, xs2 = _mid_call(parts1, xs1, b1, Wn2, Ws2, Wimp2, bimp2)
    parts2 = agg_fn(msg2, srcb, dstb, zeros)
    return _post_call(parts2, xs2, b2, Wm1, bm1, Wm2, bm2, Wm3, bm3)


# speedup vs baseline: 1.1004x; 1.1004x over previous
import functools

import jax
import jax.numpy as jnp
from jax import lax
from jax.experimental import pallas as pl
from jax.experimental.pallas import tpu as pltpu
from jax.experimental.pallas import tpu_sc as plsc

_NC = 2
_NS = 16
_NW = _NC * _NS
_CH = 120
_NBUF = 3
_IR = _NBUF + 1

_SPLIT0 = 0.63

_B = 1000


def _sigmoid(v):
    return 1.0 / (1.0 + jnp.exp(-v))


def _pre_body(x_ref, wn_ref, ws_ref, wimp_ref, bimp_ref, msg_ref, xs_ref):
    x = x_ref[...]
    xn = jnp.dot(x, wn_ref[...], preferred_element_type=jnp.float32)
    logit = jnp.dot(xn, wimp_ref[...], preferred_element_type=jnp.float32) + bimp_ref[...]
    msg_ref[...] = _sigmoid(logit) * jnp.maximum(xn, 0.0)
    xs_ref[...] = jnp.dot(x, ws_ref[...], preferred_element_type=jnp.float32)


def _mid_body(parts_ref, xs_ref, b_ref, wn_ref, ws_ref, wimp_ref, bimp_ref,
              msg_ref, xs2_ref):
    p = parts_ref[...]
    h = jnp.maximum(p[0] + p[1] + xs_ref[...] + b_ref[...], 0.0)
    xn = jnp.dot(h, wn_ref[...], preferred_element_type=jnp.float32)
    logit = jnp.dot(xn, wimp_ref[...], preferred_element_type=jnp.float32) + bimp_ref[...]
    msg_ref[...] = _sigmoid(logit) * jnp.maximum(xn, 0.0)
    xs2_ref[...] = jnp.dot(h, ws_ref[...], preferred_element_type=jnp.float32)


def _post_body(parts_ref, xs_ref, b_ref, wm1_ref, bm1_ref, wm2_ref, bm2_ref,
               wm3_ref, bm3_ref, out_ref):
    p = parts_ref[...]
    h = jnp.maximum(p[0] + p[1] + xs_ref[...] + b_ref[...], 0.0)
    m = jnp.maximum(jnp.dot(h, wm1_ref[...], preferred_element_type=jnp.float32)
                    + bm1_ref[...], 0.0)
    m = jnp.maximum(jnp.dot(m, wm2_ref[...], preferred_element_type=jnp.float32)
                    + bm2_ref[...], 0.0)
    out_ref[...] = _sigmoid(
        jnp.dot(m, wm3_ref[...], preferred_element_type=jnp.float32)
        + bm3_ref[...])


def _full(shape):
    return pl.BlockSpec(shape, lambda i: (0,) * len(shape))


def _rows(shape):
    return pl.BlockSpec(shape, lambda i: (i,) + (0,) * (len(shape) - 1))


def _parts_spec(h):
    return pl.BlockSpec((_NC, _B, h), lambda i: (0, i, 0))


def _pre_call(x, Wn, Ws, Wimp, bimp):
    n, d = x.shape
    h = Wn.shape[1]
    grid = n // _B
    return pl.pallas_call(
        _pre_body,
        grid=(grid,),
        in_specs=[_rows((_B, d)), _full((d, h)), _full((d, h)),
                  _full((d, 1)), _full((1, 1))],
        out_specs=[_rows((_B, h)), _rows((_B, h))],
        out_shape=[jax.ShapeDtypeStruct((n, h), jnp.float32),
                   jax.ShapeDtypeStruct((n, h), jnp.float32)],
    )(x, Wn, Ws, Wimp.reshape(d, 1), bimp.reshape(1, 1))


def _mid_call(parts, xs, b, Wn, Ws, Wimp, bimp):
    n, h = xs.shape
    h2 = Wn.shape[1]
    grid = n // _B
    return pl.pallas_call(
        _mid_body,
        grid=(grid,),
        in_specs=[_parts_spec(h), _rows((_B, h)), _full((1, h)),
                  _full((h, h2)), _full((h, h2)), _full((h2, 1)), _full((1, 1))],
        out_specs=[_rows((_B, h2)), _rows((_B, h2))],
        out_shape=[jax.ShapeDtypeStruct((n, h2), jnp.float32),
                   jax.ShapeDtypeStruct((n, h2), jnp.float32)],
    )(parts, xs, b.reshape(1, h), Wn, Ws, Wimp.reshape(h, 1),
      bimp.reshape(1, 1))


def _post_call(parts, xs, b, Wm1, bm1, Wm2, bm2, Wm3, bm3):
    n, h = xs.shape
    d1, d2, d3 = Wm1.shape[1], Wm2.shape[1], Wm3.shape[1]
    grid = n // _B
    return pl.pallas_call(
        _post_body,
        grid=(grid,),
        in_specs=[_parts_spec(h), _rows((_B, h)), _full((1, h)),
                  _full((h, d1)), _full((1, d1)),
                  _full((d1, d2)), _full((1, d2)),
                  _full((d2, d3)), _full((1, d3))],
        out_specs=_rows((_B, d3)),
        out_shape=jax.ShapeDtypeStruct((n, d3), jnp.float32),
    )(parts, xs, b.reshape(1, h), Wm1, bm1.reshape(1, d1),
      Wm2, bm2.reshape(1, d2), Wm3, bm3.reshape(1, d3))


@functools.lru_cache(maxsize=None)
def _make_agg(q0, q1, acc_rows, h):
    rows_per_tile = acc_rows // _NS
    mesh = plsc.VectorSubcoreMesh(core_axis_name="c", subcore_axis_name="s")

    @functools.partial(
        pl.kernel,
        mesh=mesh,
        out_type=jax.ShapeDtypeStruct((_NC, acc_rows, h), jnp.float32),
        scratch_types=(
            [pltpu.VMEM((_CH, h), jnp.float32) for _ in range(_NBUF)]
            + [pltpu.VMEM((2, _CH), jnp.int32) for _ in range(_IR)]
            + [pltpu.VMEM_SHARED((acc_rows, h), jnp.float32)]
            + [pltpu.SemaphoreType.DMA for _ in range(2 * _NBUF + _IR)]
        ),
    )
    def agg(msg_hbm, srcb_hbm, dstb_hbm, zeros_hbm, out_hbm, *scr):
        rows = scr[:_NBUF]
        ibufs = scr[_NBUF:_NBUF + _IR]
        acc = scr[_NBUF + _IR]
        rsems = scr[_NBUF + _IR + 1:2 * _NBUF + _IR + 1]
        ssems = scr[2 * _NBUF + _IR + 1:3 * _NBUF + _IR + 1]
        isems = scr[3 * _NBUF + _IR + 1:]
        c = lax.axis_index("c")
        s = lax.axis_index("s")
        base = jnp.where(c == 0, s * q0, _NS * q0 + s * q1)
        my_n = jnp.where(c == 0, q0, q1)

        def wait_rows(b):
            pltpu.make_async_copy(msg_hbm.at[pl.ds(0, _CH)], rows[b],
                                  rsems[b]).wait()

        def wait_scatter(b):
            pltpu.make_async_copy(msg_hbm.at[pl.ds(0, _CH)], rows[b],
                                  ssems[b]).wait()

        def fetch_idx(j, ib):
            pltpu.async_copy(srcb_hbm.at[j], ibufs[ib].at[0], isems[ib])
            pltpu.async_copy(dstb_hbm.at[j], ibufs[ib].at[1], isems[ib])

        def gather(b, ib):
            pltpu.make_async_copy(srcb_hbm.at[0], ibufs[ib].at[0],
                                  isems[ib]).wait()
            pltpu.make_async_copy(srcb_hbm.at[0], ibufs[ib].at[1],
                                  isems[ib]).wait()
            pltpu.async_copy(msg_hbm.at[ibufs[ib].at[0]], rows[b], rsems[b])

        for k in range(_NBUF):
            @pl.when(k < my_n)
            def _(k=k):
                fetch_idx(base + k, k)
        pltpu.sync_copy(zeros_hbm, acc.at[pl.ds(s * rows_per_tile, rows_per_tile)])
        for k in range(_NBUF - 1):
            @pl.when(k < my_n)
            def _(k=k):
                gather(k, k)
        plsc.subcore_barrier()

        def slot(j, b, ib):
            pb = (b + _NBUF - 1) % _NBUF

            @pl.when((j >= 1) & (j - 1 < my_n))
            def _():
                wait_scatter(pb)

            @pl.when(j + _NBUF - 1 < my_n)
            def _():
                gather(pb, (ib + _NBUF - 1) % _IR)

            @pl.when(j < my_n)
            def _():
                wait_rows(b)
                pltpu.async_copy(rows[b], acc.at[ibufs[ib].at[1]], ssems[b],
                                 add=True)

            @pl.when(j + _NBUF < my_n)
            def _():
                fetch_idx(base + j + _NBUF, (ib + _NBUF) % _IR)

        unroll = _NBUF * _IR

        def body(g, carry):
            for u in range(unroll):
                j = unroll * g + u

                @pl.when(j < my_n + 1)
                def _():
                    slot(j, u % _NBUF, u % _IR)
            return carry

        lax.fori_loop(0, -(-(max(q0, q1) + 1) // unroll), body, 0)
        plsc.subcore_barrier()
        pltpu.sync_copy(acc.at[pl.ds(s * rows_per_tile, rows_per_tile)],
                        out_hbm.at[c, pl.ds(s * rows_per_tile, rows_per_tile)])

    return agg


def kernel(x, edge_index, Wn1, Ws1, Wimp1, bimp1, b1, Wn2, Ws2, Wimp2, bimp2,
           b2, Wm1, bm1, Wm2, bm2, Wm3, bm3):
    n, d = x.shape
    e = edge_index.shape[1]
    h = Wn1.shape[1]

    nchunks = -(-e // _CH)
    per_pair = -(-nchunks // _NS)
    q0 = max(1, min(per_pair - 1, round(per_pair * _SPLIT0)))
    q1 = per_pair - q0
    tot = per_pair * _NS
    ep = tot * _CH
    acc_rows = (_NS * 8) * (-(-(n + 1) // (_NS * 8)))

    src = edge_index[0].astype(jnp.int32)
    dst = edge_index[1].astype(jnp.int32)
    pad = ep - e
    srcb = jnp.concatenate([src, jnp.zeros((pad,), jnp.int32)]).reshape(
        tot, _CH)
    dstb = jnp.concatenate([dst, jnp.full((pad,), n, jnp.int32)]).reshape(
        tot, _CH)
    zeros = jnp.zeros((acc_rows // _NS, h), jnp.float32)

    agg_fn = _make_agg(q0, q1, acc_rows, h)

    msg1, xs1 = _pre_call(x, Wn1, Ws1, Wimp1, bimp1)
    parts1 = agg_fn(msg1, srcb, dstb, zeros)
    msg2, xs2 = _mid_call(parts1, xs1, b1, Wn2, Ws2, Wimp2, bimp2)
    parts2 = agg_fn(msg2, srcb, dstb, zeros)
    return _post_call(parts2, xs2, b2, Wm1, bm1, Wm2, bm2, Wm3, bm3)


# speedup vs baseline: 1.1612x; 1.0553x over previous
import functools

import jax
import jax.numpy as jnp
from jax import lax
from jax.experimental import pallas as pl
from jax.experimental.pallas import tpu as pltpu
from jax.experimental.pallas import tpu_sc as plsc

_NC = 2
_NS = 16
_NW = _NC * _NS
_CH = 120
_NBUF = 3
_IR = _NBUF + 1

_SPLIT0 = 0.63

_B = 2000


def _sigmoid(v):
    return 1.0 / (1.0 + jnp.exp(-v))


def _pre_body(x_ref, wn_ref, ws_ref, wimp_ref, bimp_ref, msg_ref, xs_ref):
    x = x_ref[...]
    xn = jnp.dot(x, wn_ref[...], preferred_element_type=jnp.float32)
    logit = jnp.dot(xn, wimp_ref[...], preferred_element_type=jnp.float32) + bimp_ref[...]
    msg_ref[...] = _sigmoid(logit) * jnp.maximum(xn, 0.0)
    xs_ref[...] = jnp.dot(x, ws_ref[...], preferred_element_type=jnp.float32)


def _mid_body(parts_ref, xs_ref, b_ref, wn_ref, ws_ref, wimp_ref, bimp_ref,
              msg_ref, xs2_ref):
    p = parts_ref[...]
    h = jnp.maximum(p[0] + p[1] + xs_ref[...] + b_ref[...], 0.0)
    xn = jnp.dot(h, wn_ref[...], preferred_element_type=jnp.float32)
    logit = jnp.dot(xn, wimp_ref[...], preferred_element_type=jnp.float32) + bimp_ref[...]
    msg_ref[...] = _sigmoid(logit) * jnp.maximum(xn, 0.0)
    xs2_ref[...] = jnp.dot(h, ws_ref[...], preferred_element_type=jnp.float32)


def _post_body(parts_ref, xs_ref, b_ref, wm1_ref, bm1_ref, wm2_ref, bm2_ref,
               wm3_ref, bm3_ref, out_ref):
    p = parts_ref[...]
    h = jnp.maximum(p[0] + p[1] + xs_ref[...] + b_ref[...], 0.0)
    m = jnp.maximum(jnp.dot(h, wm1_ref[...], preferred_element_type=jnp.float32)
                    + bm1_ref[...], 0.0)
    m = jnp.maximum(jnp.dot(m, wm2_ref[...], preferred_element_type=jnp.float32)
                    + bm2_ref[...], 0.0)
    out_ref[...] = _sigmoid(
        jnp.dot(m, wm3_ref[...], preferred_element_type=jnp.float32)
        + bm3_ref[...])


def _full(shape):
    return pl.BlockSpec(shape, lambda i: (0,) * len(shape))


def _rows(shape):
    return pl.BlockSpec(shape, lambda i: (i,) + (0,) * (len(shape) - 1))


def _parts_spec(h):
    return pl.BlockSpec((_NC, _B, h), lambda i: (0, i, 0))


def _pre_call(x, Wn, Ws, Wimp, bimp):
    n, d = x.shape
    h = Wn.shape[1]
    grid = n // _B
    return pl.pallas_call(
        _pre_body,
        grid=(grid,),
        in_specs=[_rows((_B, d)), _full((d, h)), _full((d, h)),
                  _full((d, 1)), _full((1, 1))],
        out_specs=[_rows((_B, h)), _rows((_B, h))],
        out_shape=[jax.ShapeDtypeStruct((n, h), jnp.float32),
                   jax.ShapeDtypeStruct((n, h), jnp.float32)],
    )(x, Wn, Ws, Wimp.reshape(d, 1), bimp.reshape(1, 1))


def _mid_call(parts, xs, b, Wn, Ws, Wimp, bimp):
    n, h = xs.shape
    h2 = Wn.shape[1]
    grid = n // _B
    return pl.pallas_call(
        _mid_body,
        grid=(grid,),
        in_specs=[_parts_spec(h), _rows((_B, h)), _full((1, h)),
                  _full((h, h2)), _full((h, h2)), _full((h2, 1)), _full((1, 1))],
        out_specs=[_rows((_B, h2)), _rows((_B, h2))],
        out_shape=[jax.ShapeDtypeStruct((n, h2), jnp.float32),
                   jax.ShapeDtypeStruct((n, h2), jnp.float32)],
    )(parts, xs, b.reshape(1, h), Wn, Ws, Wimp.reshape(h, 1),
      bimp.reshape(1, 1))


def _post_call(parts, xs, b, Wm1, bm1, Wm2, bm2, Wm3, bm3):
    n, h = xs.shape
    d1, d2, d3 = Wm1.shape[1], Wm2.shape[1], Wm3.shape[1]
    grid = n // _B
    return pl.pallas_call(
        _post_body,
        grid=(grid,),
        in_specs=[_parts_spec(h), _rows((_B, h)), _full((1, h)),
                  _full((h, d1)), _full((1, d1)),
                  _full((d1, d2)), _full((1, d2)),
                  _full((d2, d3)), _full((1, d3))],
        out_specs=_rows((_B, d3)),
        out_shape=jax.ShapeDtypeStruct((n, d3), jnp.float32),
    )(parts, xs, b.reshape(1, h), Wm1, bm1.reshape(1, d1),
      Wm2, bm2.reshape(1, d2), Wm3, bm3.reshape(1, d3))


@functools.lru_cache(maxsize=None)
def _make_agg(q0, q1, nchunks, acc_rows, h):
    rows_per_tile = acc_rows // _NS
    mesh = plsc.VectorSubcoreMesh(core_axis_name="c", subcore_axis_name="s")

    @functools.partial(
        pl.kernel,
        mesh=mesh,
        out_type=jax.ShapeDtypeStruct((_NC, acc_rows, h), jnp.float32),
        scratch_types=(
            [pltpu.VMEM((_CH, h), jnp.float32) for _ in range(_NBUF)]
            + [pltpu.VMEM((2, _CH), jnp.int32) for _ in range(_IR)]
            + [pltpu.VMEM_SHARED((acc_rows, h), jnp.float32)]
            + [pltpu.SemaphoreType.DMA for _ in range(2 * _NBUF + _IR)]
        ),
    )
    def agg(msg_hbm, src_hbm, dst_hbm, tail_hbm, zeros_hbm, out_hbm, *scr):
        rows = scr[:_NBUF]
        ibufs = scr[_NBUF:_NBUF + _IR]
        acc = scr[_NBUF + _IR]
        rsems = scr[_NBUF + _IR + 1:2 * _NBUF + _IR + 1]
        ssems = scr[2 * _NBUF + _IR + 1:3 * _NBUF + _IR + 1]
        isems = scr[3 * _NBUF + _IR + 1:]
        c = lax.axis_index("c")
        s = lax.axis_index("s")
        base = jnp.where(c == 0, s * q0, _NS * q0 + s * q1)
        my_n = jnp.where(c == 0, q0, q1)
        my_n = jnp.maximum(0, jnp.minimum(my_n, nchunks - base))

        def wait_rows(b):
            pltpu.make_async_copy(msg_hbm.at[pl.ds(0, _CH)], rows[b],
                                  rsems[b]).wait()

        def wait_scatter(b):
            pltpu.make_async_copy(msg_hbm.at[pl.ds(0, _CH)], rows[b],
                                  ssems[b]).wait()

        def fetch_idx(g, ib):
            @pl.when(g < nchunks - 1)
            def _():
                off = pl.multiple_of(g * _CH, 8)
                pltpu.async_copy(src_hbm.at[pl.ds(off, _CH)],
                                 ibufs[ib].at[0], isems[ib])
                pltpu.async_copy(dst_hbm.at[pl.ds(off, _CH)],
                                 ibufs[ib].at[1], isems[ib])

            @pl.when(g >= nchunks - 1)
            def _():
                pltpu.async_copy(tail_hbm.at[0], ibufs[ib].at[0], isems[ib])
                pltpu.async_copy(tail_hbm.at[1], ibufs[ib].at[1], isems[ib])

        def gather(b, ib):
            pltpu.make_async_copy(src_hbm.at[pl.ds(0, _CH)], ibufs[ib].at[0],
                                  isems[ib]).wait()
            pltpu.make_async_copy(src_hbm.at[pl.ds(0, _CH)], ibufs[ib].at[1],
                                  isems[ib]).wait()
            pltpu.async_copy(msg_hbm.at[ibufs[ib].at[0]], rows[b], rsems[b])

        for k in range(_NBUF):
            @pl.when(k < my_n)
            def _(k=k):
                fetch_idx(base + k, k)
        pltpu.sync_copy(zeros_hbm, acc.at[pl.ds(s * rows_per_tile, rows_per_tile)])
        for k in range(_NBUF - 1):
            @pl.when(k < my_n)
            def _(k=k):
                gather(k, k)
        plsc.subcore_barrier()

        def slot(j, b, ib):
            pb = (b + _NBUF - 1) % _NBUF

            @pl.when((j >= 1) & (j - 1 < my_n))
            def _():
                wait_scatter(pb)

            @pl.when(j + _NBUF - 1 < my_n)
            def _():
                gather(pb, (ib + _NBUF - 1) % _IR)

            @pl.when(j < my_n)
            def _():
                wait_rows(b)
                pltpu.async_copy(rows[b], acc.at[ibufs[ib].at[1]], ssems[b],
                                 add=True)

            @pl.when(j + _NBUF < my_n)
            def _():
                fetch_idx(base + j + _NBUF, (ib + _NBUF) % _IR)

        unroll = _NBUF * _IR

        def body(g, carry):
            for u in range(unroll):
                j = unroll * g + u

                @pl.when(j < my_n + 1)
                def _():
                    slot(j, u % _NBUF, u % _IR)
            return carry

        lax.fori_loop(0, -(-(max(q0, q1) + 1) // unroll), body, 0)
        plsc.subcore_barrier()
        pltpu.sync_copy(acc.at[pl.ds(s * rows_per_tile, rows_per_tile)],
                        out_hbm.at[c, pl.ds(s * rows_per_tile, rows_per_tile)])

    return agg


def kernel(x, edge_index, Wn1, Ws1, Wimp1, bimp1, b1, Wn2, Ws2, Wimp2, bimp2,
           b2, Wm1, bm1, Wm2, bm2, Wm3, bm3):
    n, d = x.shape
    e = edge_index.shape[1]
    h = Wn1.shape[1]

    nchunks = -(-e // _CH)
    per_pair = -(-nchunks // _NS)
    q0 = max(1, min(per_pair - 1, round(per_pair * _SPLIT0)))
    q1 = per_pair - q0
    acc_rows = (_NS * 8) * (-(-(n + 1) // (_NS * 8)))

    src = edge_index[0].astype(jnp.int32)
    dst = edge_index[1].astype(jnp.int32)
    te = e - (nchunks - 1) * _CH
    tail = jnp.stack([
        jnp.concatenate([src[e - te:], jnp.zeros((_CH - te,), jnp.int32)]),
        jnp.concatenate([dst[e - te:], jnp.full((_CH - te,), n, jnp.int32)]),
    ])
    zeros = jnp.zeros((acc_rows // _NS, h), jnp.float32)

    agg_fn = _make_agg(q0, q1, nchunks, acc_rows, h)

    msg1, xs1 = _pre_call(x, Wn1, Ws1, Wimp1, bimp1)
    parts1 = agg_fn(msg1, src, dst, tail, zeros)
    msg2, xs2 = _mid_call(parts1, xs1, b1, Wn2, Ws2, Wimp2, bimp2)
    parts2 = agg_fn(msg2, src, dst, tail, zeros)
    return _post_call(parts2, xs2, b2, Wm1, bm1, Wm2, bm2, Wm3, bm3)


# speedup vs baseline: 1.3090x; 1.1272x over previous
import functools

import jax
import jax.numpy as jnp
from jax import lax
from jax.experimental import pallas as pl
from jax.experimental.pallas import tpu as pltpu
from jax.experimental.pallas import tpu_sc as plsc

_NC = 2
_NS = 16
_NW = _NC * _NS
_CH = 120
_NBUF = 3
_IR = _NBUF + 1

_SPLIT0 = 0.521

_B = 2000


def _sigmoid(v):
    return 1.0 / (1.0 + jnp.exp(-v))


def _pre_body(x_ref, wn_ref, ws_ref, wimp_ref, bimp_ref, msg_ref, xs_ref):
    x = x_ref[...]
    xn = jnp.dot(x, wn_ref[...], preferred_element_type=jnp.float32)
    logit = jnp.dot(xn, wimp_ref[...], preferred_element_type=jnp.float32) + bimp_ref[...]
    msg_ref[...] = _sigmoid(logit) * jnp.maximum(xn, 0.0)
    xs_ref[...] = jnp.dot(x, ws_ref[...], preferred_element_type=jnp.float32)


def _mid_body(parts_ref, xs_ref, b_ref, wn_ref, ws_ref, wimp_ref, bimp_ref,
              msg_ref, xs2_ref):
    p = parts_ref[...]
    h = jnp.maximum(p[0] + p[1] + xs_ref[...] + b_ref[...], 0.0)
    xn = jnp.dot(h, wn_ref[...], preferred_element_type=jnp.float32)
    logit = jnp.dot(xn, wimp_ref[...], preferred_element_type=jnp.float32) + bimp_ref[...]
    msg_ref[...] = _sigmoid(logit) * jnp.maximum(xn, 0.0)
    xs2_ref[...] = jnp.dot(h, ws_ref[...], preferred_element_type=jnp.float32)


def _post_body(parts_ref, xs_ref, b_ref, wm1_ref, bm1_ref, wm2_ref, bm2_ref,
               wm3_ref, bm3_ref, out_ref):
    p = parts_ref[...]
    h = jnp.maximum(p[0] + p[1] + xs_ref[...] + b_ref[...], 0.0)
    m = jnp.maximum(jnp.dot(h, wm1_ref[...], preferred_element_type=jnp.float32)
                    + bm1_ref[...], 0.0)
    m = jnp.maximum(jnp.dot(m, wm2_ref[...], preferred_element_type=jnp.float32)
                    + bm2_ref[...], 0.0)
    out_ref[...] = _sigmoid(
        jnp.dot(m, wm3_ref[...], preferred_element_type=jnp.float32)
        + bm3_ref[...])


def _full(shape):
    return pl.BlockSpec(shape, lambda i: (0,) * len(shape))


def _rows(shape):
    return pl.BlockSpec(shape, lambda i: (i,) + (0,) * (len(shape) - 1))


def _parts_spec(h):
    return pl.BlockSpec((_NC, _B, h), lambda i: (0, i, 0))


def _pre_call(x, Wn, Ws, Wimp, bimp):
    n, d = x.shape
    h = Wn.shape[1]
    grid = n // _B
    return pl.pallas_call(
        _pre_body,
        grid=(grid,),
        in_specs=[_rows((_B, d)), _full((d, h)), _full((d, h)),
                  _full((d, 1)), _full((1, 1))],
        out_specs=[_rows((_B, h)), _rows((_B, h))],
        out_shape=[jax.ShapeDtypeStruct((n, h), jnp.float32),
                   jax.ShapeDtypeStruct((n, h), jnp.float32)],
    )(x, Wn, Ws, Wimp.reshape(d, 1), bimp.reshape(1, 1))


def _mid_call(parts, xs, b, Wn, Ws, Wimp, bimp):
    n, h = xs.shape
    h2 = Wn.shape[1]
    grid = n // _B
    return pl.pallas_call(
        _mid_body,
        grid=(grid,),
        in_specs=[_parts_spec(h), _rows((_B, h)), _full((1, h)),
                  _full((h, h2)), _full((h, h2)), _full((h2, 1)), _full((1, 1))],
        out_specs=[_rows((_B, h2)), _rows((_B, h2))],
        out_shape=[jax.ShapeDtypeStruct((n, h2), jnp.float32),
                   jax.ShapeDtypeStruct((n, h2), jnp.float32)],
    )(parts, xs, b.reshape(1, h), Wn, Ws, Wimp.reshape(h, 1),
      bimp.reshape(1, 1))


def _post_call(parts, xs, b, Wm1, bm1, Wm2, bm2, Wm3, bm3):
    n, h = xs.shape
    d1, d2, d3 = Wm1.shape[1], Wm2.shape[1], Wm3.shape[1]
    grid = n // _B
    return pl.pallas_call(
        _post_body,
        grid=(grid,),
        in_specs=[_parts_spec(h), _rows((_B, h)), _full((1, h)),
                  _full((h, d1)), _full((1, d1)),
                  _full((d1, d2)), _full((1, d2)),
                  _full((d2, d3)), _full((1, d3))],
        out_specs=_rows((_B, d3)),
        out_shape=jax.ShapeDtypeStruct((n, d3), jnp.float32),
    )(parts, xs, b.reshape(1, h), Wm1, bm1.reshape(1, d1),
      Wm2, bm2.reshape(1, d2), Wm3, bm3.reshape(1, d3))


@functools.lru_cache(maxsize=None)
def _make_agg(q0, q1, nchunks, acc_rows, h):
    rows_per_tile = acc_rows // _NS
    mesh = plsc.VectorSubcoreMesh(core_axis_name="c", subcore_axis_name="s")

    @functools.partial(
        pl.kernel,
        mesh=mesh,
        out_type=jax.ShapeDtypeStruct((_NC, acc_rows, h), jnp.float32),
        scratch_types=(
            [pltpu.VMEM((_CH, h), jnp.float32) for _ in range(_NBUF)]
            + [pltpu.VMEM((2, _CH), jnp.int32) for _ in range(_IR)]
            + [pltpu.VMEM_SHARED((acc_rows, h), jnp.float32)]
            + [pltpu.SemaphoreType.DMA for _ in range(2 * _NBUF + _IR)]
        ),
    )
    def agg(msg_hbm, src_hbm, dst_hbm, tail_hbm, zeros_hbm, out_hbm, *scr):
        rows = scr[:_NBUF]
        ibufs = scr[_NBUF:_NBUF + _IR]
        acc = scr[_NBUF + _IR]
        rsems = scr[_NBUF + _IR + 1:2 * _NBUF + _IR + 1]
        ssems = scr[2 * _NBUF + _IR + 1:3 * _NBUF + _IR + 1]
        isems = scr[3 * _NBUF + _IR + 1:]
        c = lax.axis_index("c")
        s = lax.axis_index("s")
        base = jnp.where(c == 0, s * q0, _NS * q0 + s * q1)
        my_n = jnp.where(c == 0, q0, q1)
        my_n = jnp.maximum(0, jnp.minimum(my_n, nchunks - base))

        def wait_rows(b):
            pltpu.make_async_copy(msg_hbm.at[pl.ds(0, _CH)], rows[b],
                                  rsems[b]).wait()

        def wait_scatter(b):
            pltpu.make_async_copy(msg_hbm.at[pl.ds(0, _CH)], rows[b],
                                  ssems[b]).wait()

        def fetch_idx(g, ib):
            @pl.when(g < nchunks - 1)
            def _():
                off = pl.multiple_of(g * _CH, 8)
                pltpu.async_copy(src_hbm.at[pl.ds(off, _CH)],
                                 ibufs[ib].at[0], isems[ib])
                pltpu.async_copy(dst_hbm.at[pl.ds(off, _CH)],
                                 ibufs[ib].at[1], isems[ib])

            @pl.when(g >= nchunks - 1)
            def _():
                pltpu.async_copy(tail_hbm.at[0], ibufs[ib].at[0], isems[ib])
                pltpu.async_copy(tail_hbm.at[1], ibufs[ib].at[1], isems[ib])

        def gather(b, ib):
            pltpu.make_async_copy(src_hbm.at[pl.ds(0, _CH)], ibufs[ib].at[0],
                                  isems[ib]).wait()
            pltpu.make_async_copy(src_hbm.at[pl.ds(0, _CH)], ibufs[ib].at[1],
                                  isems[ib]).wait()
            pltpu.async_copy(msg_hbm.at[ibufs[ib].at[0]], rows[b], rsems[b])

        for k in range(_NBUF):
            @pl.when(k < my_n)
            def _(k=k):
                fetch_idx(base + k, k)
        pltpu.sync_copy(zeros_hbm, acc.at[pl.ds(s * rows_per_tile, rows_per_tile)])
        for k in range(_NBUF - 1):
            @pl.when(k < my_n)
            def _(k=k):
                gather(k, k)
        plsc.subcore_barrier()

        def slot(j, b, ib):
            pb = (b + _NBUF - 1) % _NBUF

            @pl.when((j >= 1) & (j - 1 < my_n))
            def _():
                wait_scatter(pb)

            @pl.when(j + _NBUF - 1 < my_n)
            def _():
                gather(pb, (ib + _NBUF - 1) % _IR)

            @pl.when(j < my_n)
            def _():
                wait_rows(b)
                pltpu.async_copy(rows[b], acc.at[ibufs[ib].at[1]], ssems[b],
                                 add=True)

            @pl.when(j + _NBUF < my_n)
            def _():
                fetch_idx(base + j + _NBUF, (ib + _NBUF) % _IR)

        unroll = _NBUF * _IR

        def body(g, carry):
            for u in range(unroll):
                j = unroll * g + u

                @pl.when(j < my_n + 1)
                def _():
                    slot(j, u % _NBUF, u % _IR)
            return carry

        lax.fori_loop(0, -(-(max(q0, q1) + 1) // unroll), body, 0)
        plsc.subcore_barrier()
        pltpu.sync_copy(acc.at[pl.ds(s * rows_per_tile, rows_per_tile)],
                        out_hbm.at[c, pl.ds(s * rows_per_tile, rows_per_tile)])

    return agg


def kernel(x, edge_index, Wn1, Ws1, Wimp1, bimp1, b1, Wn2, Ws2, Wimp2, bimp2,
           b2, Wm1, bm1, Wm2, bm2, Wm3, bm3):
    n, d = x.shape
    e = edge_index.shape[1]
    h = Wn1.shape[1]

    nchunks = -(-e // _CH)
    per_pair = -(-nchunks // _NS)
    q0 = max(1, min(per_pair - 1, round(per_pair * _SPLIT0)))
    q1 = per_pair - q0
    acc_rows = (_NS * 8) * (-(-(n + 1) // (_NS * 8)))

    src = edge_index[0].astype(jnp.int32)
    dst = edge_index[1].astype(jnp.int32)
    te = e - (nchunks - 1) * _CH
    tail = jnp.stack([
        jnp.concatenate([src[e - te:], jnp.zeros((_CH - te,), jnp.int32)]),
        jnp.concatenate([dst[e - te:], jnp.full((_CH - te,), n, jnp.int32)]),
    ])
    zeros = jnp.zeros((acc_rows // _NS, h), jnp.float32)

    agg_fn = _make_agg(q0, q1, nchunks, acc_rows, h)

    msg1, xs1 = _pre_call(x, Wn1, Ws1, Wimp1, bimp1)
    parts1 = agg_fn(msg1, src, dst, tail, zeros)
    msg2, xs2 = _mid_call(parts1, xs1, b1, Wn2, Ws2, Wimp2, bimp2)
    parts2 = agg_fn(msg2, src, dst, tail, zeros)
    return _post_call(parts2, xs2, b2, Wm1, bm1, Wm2, bm2, Wm3, bm3)


# speedup vs baseline: 1.3133x; 1.0033x over previous
import functools

import jax
import jax.numpy as jnp
from jax import lax
from jax.experimental import pallas as pl
from jax.experimental.pallas import tpu as pltpu
from jax.experimental.pallas import tpu_sc as plsc

_NC = 2
_NS = 16
_NW = _NC * _NS
_CH = 120
_NBUF = 3
_IR = _NBUF + 1

_SPLIT0 = 0.521

_B = 2000


def _sigmoid(v):
    return 1.0 / (1.0 + jnp.exp(-v))


def _pre_body(x_ref, wn_ref, ws_ref, wimp_ref, bimp_ref, msg_ref, xs_ref):
    x = x_ref[...]
    xn = jnp.dot(x, wn_ref[...], preferred_element_type=jnp.float32)
    logit = jnp.dot(xn, wimp_ref[...], preferred_element_type=jnp.float32) + bimp_ref[...]
    msg_ref[...] = _sigmoid(logit) * jnp.maximum(xn, 0.0)
    xs_ref[...] = jnp.dot(x, ws_ref[...], preferred_element_type=jnp.float32)


def _mid_body(parts_ref, xs_ref, b_ref, wn_ref, ws_ref, wimp_ref, bimp_ref,
              msg_ref, xs2_ref):
    p = parts_ref[...]
    h = jnp.maximum(p[0] + p[1] + xs_ref[...] + b_ref[...], 0.0)
    xn = jnp.dot(h, wn_ref[...], preferred_element_type=jnp.float32)
    logit = jnp.dot(xn, wimp_ref[...], preferred_element_type=jnp.float32) + bimp_ref[...]
    msg_ref[...] = _sigmoid(logit) * jnp.maximum(xn, 0.0)
    xs2_ref[...] = jnp.dot(h, ws_ref[...], preferred_element_type=jnp.float32)


def _post_body(parts_ref, xs_ref, b_ref, wm1_ref, bm1_ref, wm2_ref, bm2_ref,
               wm3_ref, bm3_ref, out_ref):
    p = parts_ref[...]
    h = jnp.maximum(p[0] + p[1] + xs_ref[...] + b_ref[...], 0.0)
    m = jnp.maximum(jnp.dot(h, wm1_ref[...], preferred_element_type=jnp.float32)
                    + bm1_ref[...], 0.0)
    m = jnp.maximum(jnp.dot(m, wm2_ref[...], preferred_element_type=jnp.float32)
                    + bm2_ref[...], 0.0)
    out_ref[...] = _sigmoid(
        jnp.dot(m, wm3_ref[...], preferred_element_type=jnp.float32)
        + bm3_ref[...])


def _full(shape):
    return pl.BlockSpec(shape, lambda i: (0,) * len(shape))


def _rows(shape):
    return pl.BlockSpec(shape, lambda i: (i,) + (0,) * (len(shape) - 1))


def _parts_spec(h):
    return pl.BlockSpec((_NC, _B, h), lambda i: (0, i, 0))


def _pre_call(x, Wn, Ws, Wimp, bimp):
    n, d = x.shape
    h = Wn.shape[1]
    grid = n // _B
    return pl.pallas_call(
        _pre_body,
        grid=(grid,),
        in_specs=[_rows((_B, d)), _full((d, h)), _full((d, h)),
                  _full((d, 1)), _full((1, 1))],
        out_specs=[_rows((_B, h)), _rows((_B, h))],
        out_shape=[jax.ShapeDtypeStruct((n, h), jnp.float32),
                   jax.ShapeDtypeStruct((n, h), jnp.float32)],
    )(x, Wn, Ws, Wimp.reshape(d, 1), bimp.reshape(1, 1))


def _mid_call(parts, xs, b, Wn, Ws, Wimp, bimp):
    n, h = xs.shape
    h2 = Wn.shape[1]
    grid = n // _B
    return pl.pallas_call(
        _mid_body,
        grid=(grid,),
        in_specs=[_parts_spec(h), _rows((_B, h)), _full((1, h)),
                  _full((h, h2)), _full((h, h2)), _full((h2, 1)), _full((1, 1))],
        out_specs=[_rows((_B, h2)), _rows((_B, h2))],
        out_shape=[jax.ShapeDtypeStruct((n, h2), jnp.float32),
                   jax.ShapeDtypeStruct((n, h2), jnp.float32)],
    )(parts, xs, b.reshape(1, h), Wn, Ws, Wimp.reshape(h, 1),
      bimp.reshape(1, 1))


def _post_call(parts, xs, b, Wm1, bm1, Wm2, bm2, Wm3, bm3):
    n, h = xs.shape
    d1, d2, d3 = Wm1.shape[1], Wm2.shape[1], Wm3.shape[1]
    grid = n // _B
    return pl.pallas_call(
        _post_body,
        grid=(grid,),
        in_specs=[_parts_spec(h), _rows((_B, h)), _full((1, h)),
                  _full((h, d1)), _full((1, d1)),
                  _full((d1, d2)), _full((1, d2)),
                  _full((d2, d3)), _full((1, d3))],
        out_specs=_rows((_B, d3)),
        out_shape=jax.ShapeDtypeStruct((n, d3), jnp.float32),
    )(parts, xs, b.reshape(1, h), Wm1, bm1.reshape(1, d1),
      Wm2, bm2.reshape(1, d2), Wm3, bm3.reshape(1, d3))


def _split_body(e_ref, src_ref, dst_ref):
    eb = e_ref[...]
    src_ref[...] = eb[0]
    dst_ref[...] = eb[1]


def _split_edges(edge_index):
    e = edge_index.shape[1]
    eb = 16384
    grid = -(-e // eb)
    ep = grid * eb
    return pl.pallas_call(
        _split_body,
        grid=(grid,),
        in_specs=[pl.BlockSpec((2, eb), lambda i: (0, i))],
        out_specs=[pl.BlockSpec((eb,), lambda i: (i,)),
                   pl.BlockSpec((eb,), lambda i: (i,))],
        out_shape=[jax.ShapeDtypeStruct((ep,), jnp.int32),
                   jax.ShapeDtypeStruct((ep,), jnp.int32)],
    )(edge_index)


@functools.lru_cache(maxsize=None)
def _make_agg(q0, q1, nchunks, acc_rows, h):
    rows_per_tile = acc_rows // _NS
    mesh = plsc.VectorSubcoreMesh(core_axis_name="c", subcore_axis_name="s")

    @functools.partial(
        pl.kernel,
        mesh=mesh,
        out_type=jax.ShapeDtypeStruct((_NC, acc_rows, h), jnp.float32),
        scratch_types=(
            [pltpu.VMEM((_CH, h), jnp.float32) for _ in range(_NBUF)]
            + [pltpu.VMEM((2, _CH), jnp.int32) for _ in range(_IR)]
            + [pltpu.VMEM_SHARED((acc_rows, h), jnp.float32)]
            + [pltpu.SemaphoreType.DMA for _ in range(2 * _NBUF + _IR)]
        ),
    )
    def agg(msg_hbm, src_hbm, dst_hbm, tail_hbm, zeros_hbm, out_hbm, *scr):
        rows = scr[:_NBUF]
        ibufs = scr[_NBUF:_NBUF + _IR]
        acc = scr[_NBUF + _IR]
        rsems = scr[_NBUF + _IR + 1:2 * _NBUF + _IR + 1]
        ssems = scr[2 * _NBUF + _IR + 1:3 * _NBUF + _IR + 1]
        isems = scr[3 * _NBUF + _IR + 1:]
        c = lax.axis_index("c")
        s = lax.axis_index("s")
        base = jnp.where(c == 0, s * q0, _NS * q0 + s * q1)
        my_n = jnp.where(c == 0, q0, q1)
        my_n = jnp.maximum(0, jnp.minimum(my_n, nchunks - base))

        def wait_rows(b):
            pltpu.make_async_copy(msg_hbm.at[pl.ds(0, _CH)], rows[b],
                                  rsems[b]).wait()

        def wait_scatter(b):
            pltpu.make_async_copy(msg_hbm.at[pl.ds(0, _CH)], rows[b],
                                  ssems[b]).wait()

        def fetch_idx(g, ib):
            @pl.when(g < nchunks - 1)
            def _():
                off = pl.multiple_of(g * _CH, 8)
                pltpu.async_copy(src_hbm.at[pl.ds(off, _CH)],
                                 ibufs[ib].at[0], isems[ib])
                pltpu.async_copy(dst_hbm.at[pl.ds(off, _CH)],
                                 ibufs[ib].at[1], isems[ib])

            @pl.when(g >= nchunks - 1)
            def _():
                pltpu.async_copy(tail_hbm.at[0], ibufs[ib].at[0], isems[ib])
                pltpu.async_copy(tail_hbm.at[1], ibufs[ib].at[1], isems[ib])

        def gather(b, ib):
            pltpu.make_async_copy(src_hbm.at[pl.ds(0, _CH)], ibufs[ib].at[0],
                                  isems[ib]).wait()
            pltpu.make_async_copy(src_hbm.at[pl.ds(0, _CH)], ibufs[ib].at[1],
                                  isems[ib]).wait()
            pltpu.async_copy(msg_hbm.at[ibufs[ib].at[0]], rows[b], rsems[b])

        for k in range(_NBUF):
            @pl.when(k < my_n)
            def _(k=k):
                fetch_idx(base + k, k)
        pltpu.sync_copy(zeros_hbm, acc.at[pl.ds(s * rows_per_tile, rows_per_tile)])
        for k in range(_NBUF - 1):
            @pl.when(k < my_n)
            def _(k=k):
                gather(k, k)
        plsc.subcore_barrier()

        def slot(j, b, ib):
            pb = (b + _NBUF - 1) % _NBUF

            @pl.when((j >= 1) & (j - 1 < my_n))
            def _():
                wait_scatter(pb)

            @pl.when(j + _NBUF - 1 < my_n)
            def _():
                gather(pb, (ib + _NBUF - 1) % _IR)

            @pl.when(j < my_n)
            def _():
                wait_rows(b)
                pltpu.async_copy(rows[b], acc.at[ibufs[ib].at[1]], ssems[b],
                                 add=True)

            @pl.when(j + _NBUF < my_n)
            def _():
                fetch_idx(base + j + _NBUF, (ib + _NBUF) % _IR)

        unroll = _NBUF * _IR

        def body(g, carry):
            for u in range(unroll):
                j = unroll * g + u

                @pl.when(j < my_n + 1)
                def _():
                    slot(j, u % _NBUF, u % _IR)
            return carry

        lax.fori_loop(0, -(-(max(q0, q1) + 1) // unroll), body, 0)
        plsc.subcore_barrier()
        pltpu.sync_copy(acc.at[pl.ds(s * rows_per_tile, rows_per_tile)],
                        out_hbm.at[c, pl.ds(s * rows_per_tile, rows_per_tile)])

    return agg


def kernel(x, edge_index, Wn1, Ws1, Wimp1, bimp1, b1, Wn2, Ws2, Wimp2, bimp2,
           b2, Wm1, bm1, Wm2, bm2, Wm3, bm3):
    n, d = x.shape
    e = edge_index.shape[1]
    h = Wn1.shape[1]

    nchunks = -(-e // _CH)
    per_pair = -(-nchunks // _NS)
    q0 = max(1, min(per_pair - 1, round(per_pair * _SPLIT0)))
    q1 = per_pair - q0
    acc_rows = (_NS * 8) * (-(-(n + 1) // (_NS * 8)))

    src, dst = _split_edges(edge_index.astype(jnp.int32))
    te = e - (nchunks - 1) * _CH
    tail = jnp.stack([
        jnp.concatenate([edge_index[0, e - te:].astype(jnp.int32),
                         jnp.zeros((_CH - te,), jnp.int32)]),
        jnp.concatenate([edge_index[1, e - te:].astype(jnp.int32),
                         jnp.full((_CH - te,), n, jnp.int32)]),
    ])
    zeros = jnp.zeros((acc_rows // _NS, h), jnp.float32)

    agg_fn = _make_agg(q0, q1, nchunks, acc_rows, h)

    msg1, xs1 = _pre_call(x, Wn1, Ws1, Wimp1, bimp1)
    parts1 = agg_fn(msg1, src, dst, tail, zeros)
    msg2, xs2 = _mid_call(parts1, xs1, b1, Wn2, Ws2, Wimp2, bimp2)
    parts2 = agg_fn(msg2, src, dst, tail, zeros)
    return _post_call(parts2, xs2, b2, Wm1, bm1, Wm2, bm2, Wm3, bm3)


# speedup vs baseline: 1.3284x; 1.0115x over previous
import functools

import jax
import jax.numpy as jnp
from jax import lax
from jax.experimental import pallas as pl
from jax.experimental.pallas import tpu as pltpu
from jax.experimental.pallas import tpu_sc as plsc

_NC = 2
_NS = 16
_NW = _NC * _NS
_CH = 120
_NBUF = 3
_IR = _NBUF + 1

_SPLIT0 = 0.503

_B = 2000


def _sigmoid(v):
    return 1.0 / (1.0 + jnp.exp(-v))


def _pre_body(x_ref, wn_ref, ws_ref, wimp_ref, bimp_ref, msg_ref, xs_ref):
    x = x_ref[...]
    xn = jnp.dot(x, wn_ref[...], preferred_element_type=jnp.float32)
    logit = jnp.dot(xn, wimp_ref[...], preferred_element_type=jnp.float32) + bimp_ref[...]
    msg_ref[...] = _sigmoid(logit) * jnp.maximum(xn, 0.0)
    xs_ref[...] = jnp.dot(x, ws_ref[...], preferred_element_type=jnp.float32)


def _mid_body(parts_ref, xs_ref, b_ref, wn_ref, ws_ref, wimp_ref, bimp_ref,
              msg_ref, xs2_ref):
    p = parts_ref[...]
    h = jnp.maximum(p[0] + p[1] + xs_ref[...] + b_ref[...], 0.0)
    xn = jnp.dot(h, wn_ref[...], preferred_element_type=jnp.float32)
    logit = jnp.dot(xn, wimp_ref[...], preferred_element_type=jnp.float32) + bimp_ref[...]
    msg_ref[...] = _sigmoid(logit) * jnp.maximum(xn, 0.0)
    xs2_ref[...] = jnp.dot(h, ws_ref[...], preferred_element_type=jnp.float32)


def _post_body(parts_ref, xs_ref, b_ref, wm1_ref, bm1_ref, wm2_ref, bm2_ref,
               wm3_ref, bm3_ref, out_ref):
    p = parts_ref[...]
    h = jnp.maximum(p[0] + p[1] + xs_ref[...] + b_ref[...], 0.0)
    m = jnp.maximum(jnp.dot(h, wm1_ref[...], preferred_element_type=jnp.float32)
                    + bm1_ref[...], 0.0)
    m = jnp.maximum(jnp.dot(m, wm2_ref[...], preferred_element_type=jnp.float32)
                    + bm2_ref[...], 0.0)
    out_ref[...] = _sigmoid(
        jnp.dot(m, wm3_ref[...], preferred_element_type=jnp.float32)
        + bm3_ref[...])


def _full(shape):
    return pl.BlockSpec(shape, lambda i: (0,) * len(shape))


def _rows(shape):
    return pl.BlockSpec(shape, lambda i: (i,) + (0,) * (len(shape) - 1))


def _parts_spec(h):
    return pl.BlockSpec((_NC, _B, h), lambda i: (0, i, 0))


def _pre_call(x, Wn, Ws, Wimp, bimp):
    n, d = x.shape
    h = Wn.shape[1]
    grid = n // _B
    return pl.pallas_call(
        _pre_body,
        grid=(grid,),
        in_specs=[_rows((_B, d)), _full((d, h)), _full((d, h)),
                  _full((d, 1)), _full((1, 1))],
        out_specs=[_rows((_B, h)), _rows((_B, h))],
        out_shape=[jax.ShapeDtypeStruct((n, h), jnp.float32),
                   jax.ShapeDtypeStruct((n, h), jnp.float32)],
    )(x, Wn, Ws, Wimp.reshape(d, 1), bimp.reshape(1, 1))


def _mid_call(parts, xs, b, Wn, Ws, Wimp, bimp):
    n, h = xs.shape
    h2 = Wn.shape[1]
    grid = n // _B
    return pl.pallas_call(
        _mid_body,
        grid=(grid,),
        in_specs=[_parts_spec(h), _rows((_B, h)), _full((1, h)),
                  _full((h, h2)), _full((h, h2)), _full((h2, 1)), _full((1, 1))],
        out_specs=[_rows((_B, h2)), _rows((_B, h2))],
        out_shape=[jax.ShapeDtypeStruct((n, h2), jnp.float32),
                   jax.ShapeDtypeStruct((n, h2), jnp.float32)],
    )(parts, xs, b.reshape(1, h), Wn, Ws, Wimp.reshape(h, 1),
      bimp.reshape(1, 1))


def _post_call(parts, xs, b, Wm1, bm1, Wm2, bm2, Wm3, bm3):
    n, h = xs.shape
    d1, d2, d3 = Wm1.shape[1], Wm2.shape[1], Wm3.shape[1]
    grid = n // _B
    return pl.pallas_call(
        _post_body,
        grid=(grid,),
        in_specs=[_parts_spec(h), _rows((_B, h)), _full((1, h)),
                  _full((h, d1)), _full((1, d1)),
                  _full((d1, d2)), _full((1, d2)),
                  _full((d2, d3)), _full((1, d3))],
        out_specs=_rows((_B, d3)),
        out_shape=jax.ShapeDtypeStruct((n, d3), jnp.float32),
    )(parts, xs, b.reshape(1, h), Wm1, bm1.reshape(1, d1),
      Wm2, bm2.reshape(1, d2), Wm3, bm3.reshape(1, d3))


def _split_body(e_ref, src_ref, dst_ref):
    eb = e_ref[...]
    src_ref[...] = eb[0]
    dst_ref[...] = eb[1]


def _split_edges(edge_index):
    e = edge_index.shape[1]
    eb = 16384
    grid = -(-e // eb)
    ep = grid * eb
    return pl.pallas_call(
        _split_body,
        grid=(grid,),
        in_specs=[pl.BlockSpec((2, eb), lambda i: (0, i))],
        out_specs=[pl.BlockSpec((eb,), lambda i: (i,)),
                   pl.BlockSpec((eb,), lambda i: (i,))],
        out_shape=[jax.ShapeDtypeStruct((ep,), jnp.int32),
                   jax.ShapeDtypeStruct((ep,), jnp.int32)],
    )(edge_index)


@functools.lru_cache(maxsize=None)
def _make_agg(q0, q1, nchunks, acc_rows, h):
    rows_per_tile = acc_rows // _NS
    mesh = plsc.VectorSubcoreMesh(core_axis_name="c", subcore_axis_name="s")

    @functools.partial(
        pl.kernel,
        mesh=mesh,
        out_type=jax.ShapeDtypeStruct((_NC, acc_rows, h), jnp.float32),
        scratch_types=(
            [pltpu.VMEM((_CH, h), jnp.float32) for _ in range(_NBUF)]
            + [pltpu.VMEM((2, _CH), jnp.int32) for _ in range(_IR)]
            + [pltpu.VMEM_SHARED((acc_rows, h), jnp.float32)]
            + [pltpu.SemaphoreType.DMA for _ in range(2 * _NBUF + _IR)]
        ),
    )
    def agg(msg_hbm, src_hbm, dst_hbm, tail_hbm, zeros_hbm, out_hbm, *scr):
        rows = scr[:_NBUF]
        ibufs = scr[_NBUF:_NBUF + _IR]
        acc = scr[_NBUF + _IR]
        rsems = scr[_NBUF + _IR + 1:2 * _NBUF + _IR + 1]
        ssems = scr[2 * _NBUF + _IR + 1:3 * _NBUF + _IR + 1]
        isems = scr[3 * _NBUF + _IR + 1:]
        c = lax.axis_index("c")
        s = lax.axis_index("s")
        base = jnp.where(c == 0, s * q0, _NS * q0 + s * q1)
        my_n = jnp.where(c == 0, q0, q1)
        my_n = jnp.maximum(0, jnp.minimum(my_n, nchunks - base))

        def wait_rows(b):
            pltpu.make_async_copy(msg_hbm.at[pl.ds(0, _CH)], rows[b],
                                  rsems[b]).wait()

        def wait_scatter(b):
            pltpu.make_async_copy(msg_hbm.at[pl.ds(0, _CH)], rows[b],
                                  ssems[b]).wait()

        def fetch_idx(g, ib):
            @pl.when(g < nchunks - 1)
            def _():
                off = pl.multiple_of(g * _CH, 8)
                pltpu.async_copy(src_hbm.at[pl.ds(off, _CH)],
                                 ibufs[ib].at[0], isems[ib])
                pltpu.async_copy(dst_hbm.at[pl.ds(off, _CH)],
                                 ibufs[ib].at[1], isems[ib])

            @pl.when(g >= nchunks - 1)
            def _():
                pltpu.async_copy(tail_hbm.at[0], ibufs[ib].at[0], isems[ib])
                pltpu.async_copy(tail_hbm.at[1], ibufs[ib].at[1], isems[ib])

        def gather(b, ib):
            pltpu.make_async_copy(src_hbm.at[pl.ds(0, _CH)], ibufs[ib].at[0],
                                  isems[ib]).wait()
            pltpu.make_async_copy(src_hbm.at[pl.ds(0, _CH)], ibufs[ib].at[1],
                                  isems[ib]).wait()
            pltpu.async_copy(msg_hbm.at[ibufs[ib].at[0]], rows[b], rsems[b])

        for k in range(_NBUF):
            @pl.when(k < my_n)
            def _(k=k):
                fetch_idx(base + k, k)
        pltpu.sync_copy(zeros_hbm, acc.at[pl.ds(s * rows_per_tile, rows_per_tile)])
        for k in range(_NBUF - 1):
            @pl.when(k < my_n)
            def _(k=k):
                gather(k, k)
        plsc.subcore_barrier()

        def slot(j, b, ib):
            pb = (b + _NBUF - 1) % _NBUF

            @pl.when((j >= 1) & (j - 1 < my_n))
            def _():
                wait_scatter(pb)

            @pl.when(j + _NBUF - 1 < my_n)
            def _():
                gather(pb, (ib + _NBUF - 1) % _IR)

            @pl.when(j < my_n)
            def _():
                wait_rows(b)
                pltpu.async_copy(rows[b], acc.at[ibufs[ib].at[1]], ssems[b],
                                 add=True)

            @pl.when(j + _NBUF < my_n)
            def _():
                fetch_idx(base + j + _NBUF, (ib + _NBUF) % _IR)

        unroll = _NBUF * _IR

        def body(g, carry):
            for u in range(unroll):
                j = unroll * g + u

                @pl.when(j < my_n + 1)
                def _():
                    slot(j, u % _NBUF, u % _IR)
            return carry

        lax.fori_loop(0, -(-(max(q0, q1) + 1) // unroll), body, 0)
        plsc.subcore_barrier()
        pltpu.sync_copy(acc.at[pl.ds(s * rows_per_tile, rows_per_tile)],
                        out_hbm.at[c, pl.ds(s * rows_per_tile, rows_per_tile)])

    return agg


def kernel(x, edge_index, Wn1, Ws1, Wimp1, bimp1, b1, Wn2, Ws2, Wimp2, bimp2,
           b2, Wm1, bm1, Wm2, bm2, Wm3, bm3):
    n, d = x.shape
    e = edge_index.shape[1]
    h = Wn1.shape[1]

    nchunks = -(-e // _CH)
    per_pair = -(-nchunks // _NS)
    q0 = max(1, min(per_pair - 1, round(per_pair * _SPLIT0)))
    q1 = per_pair - q0
    acc_rows = (_NS * 8) * (-(-(n + 1) // (_NS * 8)))

    src, dst = _split_edges(edge_index.astype(jnp.int32))
    te = e - (nchunks - 1) * _CH
    tail = jnp.stack([
        jnp.concatenate([edge_index[0, e - te:].astype(jnp.int32),
                         jnp.zeros((_CH - te,), jnp.int32)]),
        jnp.concatenate([edge_index[1, e - te:].astype(jnp.int32),
                         jnp.full((_CH - te,), n, jnp.int32)]),
    ])
    zeros = jnp.zeros((acc_rows // _NS, h), jnp.float32)

    agg_fn = _make_agg(q0, q1, nchunks, acc_rows, h)

    msg1, xs1 = _pre_call(x, Wn1, Ws1, Wimp1, bimp1)
    parts1 = agg_fn(msg1, src, dst, tail, zeros)
    msg2, xs2 = _mid_call(parts1, xs1, b1, Wn2, Ws2, Wimp2, bimp2)
    parts2 = agg_fn(msg2, src, dst, tail, zeros)
    return _post_call(parts2, xs2, b2, Wm1, bm1, Wm2, bm2, Wm3, bm3)
